# Initial kernel scaffold; baseline (speedup 1.0000x reference)
import jax
import jax.numpy as jnp
from jax.experimental import pallas as pl


def kernel(x, edge_index, W1, b1, W2, b2, W3, b3, Wf1, bf1, Wf2, bf2, Wis, bis, Wmc, bmc):
    raise NotImplementedError("write your pallas kernel here")


# speedup vs baseline: 8.7107x; 8.7107x over previous
import functools

import jax
import jax.numpy as jnp
from jax import lax
from jax.experimental import pallas as pl
from jax.experimental.pallas import tpu as pltpu
from jax.experimental.pallas import tpu_sc as plsc

N = 50000
NP = 51200
E = 800000
EP = 851968
CH = 128
GRP = 16
NCHUNK = EP // CH
ROWS_T = NP // 16
ZR = 640
BR = 1000
GRID = N // BR

_f32 = jnp.float32


def _zero_acc(zeros_hbm, acc, s):
    def zb(k, carry):
        pltpu.sync_copy(zeros_hbm, acc.at[pl.ds(s * ROWS_T + k * ZR, ZR)])
        return carry
    lax.fori_loop(0, ROWS_T // ZR, zb, 0)


def _writeback(acc, out, s):
    pltpu.sync_copy(acc.at[pl.ds(s * ROWS_T, ROWS_T)],
                    out.at[pl.ds(s * ROWS_T, ROWS_T)])


def _acc_loop(src2, dst2, table, src_b, dst_b, rows_b, acc, sem, base, ngrp):
    def grp_body(g, carry):
        r0 = base + g * GRP
        pltpu.sync_copy(src2.at[pl.ds(r0, GRP)], src_b)
        pltpu.sync_copy(dst2.at[pl.ds(r0, GRP)], dst_b)
        for j in range(GRP):
            pltpu.async_copy(table.at[src_b.at[j]], rows_b, sem).wait()
            pltpu.sync_copy(rows_b, acc.at[dst_b.at[j]], add=True)
        return carry
    lax.fori_loop(0, ngrp, grp_body, 0)


def _make_deg():
    W = 16
    mesh = plsc.VectorSubcoreMesh(core_axis_name="c", subcore_axis_name="s")

    @functools.partial(
        pl.kernel, mesh=mesh,
        compiler_params=pltpu.CompilerParams(use_tc_tiling_on_sc=False),
        out_type=[jax.ShapeDtypeStruct((NP, W), _f32),
                  jax.ShapeDtypeStruct((NP, W), _f32)],
        scratch_types=[pltpu.VMEM((GRP, CH), jnp.int32),
                       pltpu.VMEM((CH, W), _f32),
                       pltpu.VMEM_SHARED((NP, W), _f32)],
    )
    def deg_k(dst2, ones_hbm, zeros_hbm, out_a, out_b, dst_b, ones_b, acc):
        c = lax.axis_index("c")
        s = lax.axis_index("s")
        pltpu.sync_copy(ones_hbm, ones_b)
        _zero_acc(zeros_hbm, acc, s)
        plsc.subcore_barrier()
        base = (c * 16 + s) * (NCHUNK // 32)

        def grp_body(g, carry):
            r0 = base + g * GRP
            pltpu.sync_copy(dst2.at[pl.ds(r0, GRP)], dst_b)
            for j in range(GRP):
                pltpu.sync_copy(ones_b, acc.at[dst_b.at[j]], add=True)
            return carry
        lax.fori_loop(0, (NCHUNK // 32) // GRP, grp_body, 0)
        plsc.subcore_barrier()

        @pl.when(c == 0)
        def _():
            _writeback(acc, out_a, s)

        @pl.when(c == 1)
        def _():
            _writeback(acc, out_b, s)

    return deg_k


def _make_prop16():
    W = 16
    mesh = plsc.VectorSubcoreMesh(core_axis_name="c", subcore_axis_name="s")

    @functools.partial(
        pl.kernel, mesh=mesh,
        compiler_params=pltpu.CompilerParams(use_tc_tiling_on_sc=False),
        out_type=[jax.ShapeDtypeStruct((NP, W), _f32),
                  jax.ShapeDtypeStruct((NP, W), _f32)],
        scratch_types=[pltpu.VMEM((GRP, CH), jnp.int32),
                       pltpu.VMEM((GRP, CH), jnp.int32),
                       pltpu.VMEM((CH, W), _f32),
                       pltpu.VMEM_SHARED((NP, W), _f32),
                       pltpu.SemaphoreType.DMA],
    )
    def prop_k(src2, dst2, table, zeros_hbm, out_a, out_b,
               src_b, dst_b, rows_b, acc, sem):
        c = lax.axis_index("c")
        s = lax.axis_index("s")
        _zero_acc(zeros_hbm, acc, s)
        plsc.subcore_barrier()
        base = (c * 16 + s) * (NCHUNK // 32)
        _acc_loop(src2, dst2, table, src_b, dst_b, rows_b, acc, sem,
                  base, (NCHUNK // 32) // GRP)
        plsc.subcore_barrier()

        @pl.when(c == 0)
        def _():
            _writeback(acc, out_a, s)

        @pl.when(c == 1)
        def _():
            _writeback(acc, out_b, s)

    return prop_k


def _make_prop32():
    W = 32
    mesh = plsc.VectorSubcoreMesh(core_axis_name="c", subcore_axis_name="s")

    @functools.partial(
        pl.kernel, mesh=mesh,
        compiler_params=pltpu.CompilerParams(use_tc_tiling_on_sc=False),
        out_type=[jax.ShapeDtypeStruct((NP, W), _f32),
                  jax.ShapeDtypeStruct((NP, W), _f32)],
        scratch_types=[pltpu.VMEM((GRP, CH), jnp.int32),
                       pltpu.VMEM((GRP, CH), jnp.int32),
                       pltpu.VMEM((CH, W), _f32),
                       pltpu.VMEM_SHARED((NP, W), _f32),
                       pltpu.SemaphoreType.DMA],
    )
    def prop_k(src2, dst2, t_lo, t_hi, zeros_hbm, out_lo, out_hi,
               src_b, dst_b, rows_b, acc, sem):
        c = lax.axis_index("c")
        s = lax.axis_index("s")
        _zero_acc(zeros_hbm, acc, s)
        plsc.subcore_barrier()
        base = s * (NCHUNK // 16)
        ngrp = (NCHUNK // 16) // GRP

        @pl.when(c == 0)
        def _():
            _acc_loop(src2, dst2, t_lo, src_b, dst_b, rows_b, acc, sem,
                      base, ngrp)

        @pl.when(c == 1)
        def _():
            _acc_loop(src2, dst2, t_hi, src_b, dst_b, rows_b, acc, sem,
                      base, ngrp)
        plsc.subcore_barrier()

        @pl.when(c == 0)
        def _():
            _writeback(acc, out_lo, s)

        @pl.when(c == 1)
        def _():
            _writeback(acc, out_hi, s)

    return prop_k


_deg = _make_deg()
_prop16 = _make_prop16()
_prop32 = _make_prop32()


def _row_spec(w):
    return pl.BlockSpec((BR, w), lambda i: (i, 0))


def _full_spec(shape):
    return pl.BlockSpec(shape, lambda i: (0, 0))


def _prep(deg_a, deg_b, xp):
    def body(da, db, xr, g0, dinv):
        deg = da[:, 0:1] + db[:, 0:1] + 1.0
        di = lax.rsqrt(deg)
        dinv[...] = di
        g0[...] = xr[...] * di

    return pl.pallas_call(
        body, grid=(GRID,),
        in_specs=[_row_spec(16), _row_spec(16), _row_spec(16)],
        out_specs=[_row_spec(16), _row_spec(1)],
        out_shape=[jax.ShapeDtypeStruct((N, 16), _f32),
                   jax.ShapeDtypeStruct((N, 1), _f32)],
    )(deg_a, deg_b, xp)


def _layer1(a0a, a0b, g0, dinv, w1p, b1, w2):
    def body(aa, ab, g0r, dr, w1r, b1r, w2r, lo, hi):
        di = dr[...]
        q = di * (aa[...] + ab[...] + g0r[...])
        h1 = jnp.maximum(
            jnp.dot(q, w1r[...], preferred_element_type=_f32) + b1r[...], 0.0)
        g1 = di * jnp.dot(h1, w2r[...], preferred_element_type=_f32)
        lo[...] = g1[:, :32]
        hi[...] = g1[:, 32:]

    return pl.pallas_call(
        body, grid=(GRID,),
        in_specs=[_row_spec(16), _row_spec(16), _row_spec(16), _row_spec(1),
                  _full_spec((16, 64)), _full_spec((1, 64)),
                  _full_spec((64, 64))],
        out_specs=[_row_spec(32), _row_spec(32)],
        out_shape=[jax.ShapeDtypeStruct((N, 32), _f32),
                   jax.ShapeDtypeStruct((N, 32), _f32)],
    )(a0a, a0b, g0, dinv, w1p, b1, w2)


def _layer_mid(alo, ahi, glo, ghi, dinv, b, w):
    def body(alr, ahr, glr, ghr, dr, br, wr, lo, hi):
        di = dr[...]
        h = jnp.concatenate([alr[...] + glr[...], ahr[...] + ghr[...]],
                            axis=1)
        h = jnp.maximum(di * h + br[...], 0.0)
        g = di * jnp.dot(h, wr[...], preferred_element_type=_f32)
        lo[...] = g[:, :32]
        hi[...] = g[:, 32:]

    return pl.pallas_call(
        body, grid=(GRID,),
        in_specs=[_row_spec(32), _row_spec(32), _row_spec(32), _row_spec(32),
                  _row_spec(1), _full_spec((1, 64)), _full_spec((64, 64))],
        out_specs=[_row_spec(32), _row_spec(32)],
        out_shape=[jax.ShapeDtypeStruct((N, 32), _f32),
                   jax.ShapeDtypeStruct((N, 32), _f32)],
    )(alo, ahi, glo, ghi, dinv, b, w)


def _final(alo, ahi, glo, ghi, dinv, b3, xp, wf1p, bf1, wf2, bf2, wcat, bcat):
    def body(alr, ahr, glr, ghr, dr, b3r, xr, wf1r, bf1r, wf2r, bf2r,
             wcr, bcr, out, emb_s, gfs):
        i = pl.program_id(0)

        @pl.when(i == 0)
        def _():
            emb_s[...] = jnp.zeros_like(emb_s)
            for k in range(6):
                gfs[k] = 0.0

        di = dr[...]
        h = jnp.concatenate([alr[...] + glr[...], ahr[...] + ghr[...]],
                            axis=1)
        h3 = jnp.maximum(di * h + b3r[...], 0.0)
        emb_s[...] = emb_s[...] + jnp.sum(h3, axis=0, keepdims=True)

        xb = xr[...]
        x2 = xb[:, 2:3]
        m = x2 == 1.0
        gfs[0] = gfs[0] + jnp.sum(x2)
        gfs[1] = gfs[1] + jnp.sum(xb[:, 3:4])
        gfs[2] = gfs[2] + jnp.sum(xb[:, 4:5])
        gfs[3] = gfs[3] + jnp.sum(m.astype(_f32))
        gfs[4] = gfs[4] + jnp.sum(jnp.where(m, xb[:, 0:1], 0.0))
        gfs[5] = gfs[5] + jnp.sum(jnp.where(m, xb[:, 1:2], 0.0))

        @pl.when(i == GRID - 1)
        def _():
            emb = emb_s[...] * (1.0 / N)
            n_comp, n_and, n_or = gfs[0], gfs[1], gfs[2]
            cnt, s0, s1 = gfs[3], gfs[4], gfs[5]
            avg_l = jnp.where(cnt > 0, s0 / jnp.maximum(cnt, 1.0), 0.0)
            avg_m = jnp.where(cnt > 0, s1 / jnp.maximum(cnt, 1.0), 0.0)
            lane = lax.broadcasted_iota(jnp.int32, (1, 8), 1)
            z = jnp.zeros((1, 8), _f32)
            gf = (jnp.where(lane == 0, n_comp, z)
                  + jnp.where(lane == 1, n_and, z)
                  + jnp.where(lane == 2, n_or, z)
                  + jnp.where(lane == 3, n_and + n_or, z)
                  + jnp.where(lane == 4, avg_l, z)
                  + jnp.where(lane == 5, avg_m, z)
                  + jnp.where(lane == 6, jnp.float32(0.2), z))
            f1 = jnp.maximum(
                jnp.dot(emb, wf1r[0:64, :], preferred_element_type=_f32)
                + jnp.dot(gf, wf1r[64:72, :], preferred_element_type=_f32)
                + bf1r[...], 0.0)
            f2 = jnp.maximum(
                jnp.dot(f1, wf2r[...], preferred_element_type=_f32)
                + bf2r[...], 0.0)
            out[...] = (jnp.dot(f2, wcr[...], preferred_element_type=_f32)
                        + bcr[...])

    return pl.pallas_call(
        body, grid=(GRID,),
        in_specs=[_row_spec(32), _row_spec(32), _row_spec(32), _row_spec(32),
                  _row_spec(1), _full_spec((1, 64)), _row_spec(16),
                  _full_spec((72, 64)), _full_spec((1, 64)),
                  _full_spec((64, 32)), _full_spec((1, 32)),
                  _full_spec((32, 16)), _full_spec((1, 16))],
        out_specs=pl.BlockSpec((1, 16), lambda i: (0, 0)),
        out_shape=jax.ShapeDtypeStruct((1, 16), _f32),
        scratch_shapes=[pltpu.VMEM((1, 64), _f32),
                        pltpu.SMEM((8,), _f32)],
    )(alo, ahi, glo, ghi, dinv, b3, xp, wf1p, bf1, wf2, bf2, wcat, bcat)


def kernel(x, edge_index, W1, b1, W2, b2, W3, b3,
           Wf1, bf1, Wf2, bf2, Wis, bis, Wmc, bmc):
    xp = jnp.pad(x, ((0, 0), (0, 11)))
    src = jnp.concatenate(
        [edge_index[0], jnp.zeros((EP - E,), jnp.int32)])
    pad_dst = N + (jnp.arange(EP - E, dtype=jnp.int32) % (NP - N))
    dst = jnp.concatenate([edge_index[1], pad_dst])
    src2 = src.reshape(NCHUNK, CH)
    dst2 = dst.reshape(NCHUNK, CH)
    ones16 = jnp.ones((CH, 16), _f32)
    zeros16 = jnp.zeros((ZR, 16), _f32)
    zeros32 = jnp.zeros((ZR, 32), _f32)

    deg_a, deg_b = _deg(dst2, ones16, zeros16)
    g0, dinv = _prep(deg_a, deg_b, xp)
    a0a, a0b = _prop16(src2, dst2, g0, zeros16)
    w1p = jnp.pad(W1, ((0, 11), (0, 0)))
    g1lo, g1hi = _layer1(a0a, a0b, g0, dinv, w1p, b1.reshape(1, -1), W2)
    a1lo, a1hi = _prop32(src2, dst2, g1lo, g1hi, zeros32)
    g2lo, g2hi = _layer_mid(a1lo, a1hi, g1lo, g1hi, dinv,
                            b2.reshape(1, -1), W3)
    a2lo, a2hi = _prop32(src2, dst2, g2lo, g2hi, zeros32)
    wf1p = jnp.pad(Wf1, ((0, 1), (0, 0)))
    wcat = jnp.concatenate([jnp.pad(Wis, ((0, 0), (0, 2))),
                            jnp.pad(Wmc, ((0, 0), (0, 2)))], axis=1)
    bcat = jnp.concatenate([jnp.pad(bis, (0, 2)),
                            jnp.pad(bmc, (0, 2))]).reshape(1, 16)
    out = _final(a2lo, a2hi, g2lo, g2hi, dinv, b3.reshape(1, -1), xp,
                 wf1p, bf1.reshape(1, -1), Wf2, bf2.reshape(1, -1),
                 wcat, bcat)
    return out[:, 0:6], out[:, 8:14]


# speedup vs baseline: 8.9716x; 1.0299x over previous
import functools

import jax
import jax.numpy as jnp
from jax import lax
from jax.experimental import pallas as pl
from jax.experimental.pallas import tpu as pltpu
from jax.experimental.pallas import tpu_sc as plsc

N = 50000
NP = 51200
E = 800000
EP = 851968
CH = 128
GRP = 8
NCHUNK = EP // CH
ROWS_T = NP // 16
ZR = 640
BR = 1000
GRID = N // BR

_f32 = jnp.float32


def _zero_acc(zeros_hbm, acc, s):
    def zb(k, carry):
        pltpu.sync_copy(zeros_hbm, acc.at[pl.ds(s * ROWS_T + k * ZR, ZR)])
        return carry
    lax.fori_loop(0, ROWS_T // ZR, zb, 0)


def _writeback(acc, out, s):
    pltpu.sync_copy(acc.at[pl.ds(s * ROWS_T, ROWS_T)],
                    out.at[pl.ds(s * ROWS_T, ROWS_T)])


def _drain(zeros_hbm, buf, sem):
    pltpu.make_async_copy(zeros_hbm.at[pl.ds(0, CH)], buf, sem).wait()


def _acc_ring(src2, dst2, table, zeros_hbm, src_i, dst_i, bufs, acc,
              gsem, ssem, base, ngrp):
    pltpu.sync_copy(src2.at[pl.ds(base, GRP)], src_i.at[0])
    pltpu.sync_copy(dst2.at[pl.ds(base, GRP)], dst_i.at[0])
    pltpu.async_copy(table.at[src_i.at[0, 0]], bufs.at[0], gsem)
    pltpu.async_copy(zeros_hbm.at[pl.ds(0, CH)], bufs.at[1], ssem)

    def body(g, carry):
        pg = lax.rem(g, 2)
        qg = lax.rem(g + 1, 2)
        for b in range(GRP):
            pbuf = b % 2
            qbuf = (b + 1) % 2
            _drain(zeros_hbm, bufs.at[pbuf], gsem)
            _drain(zeros_hbm, bufs.at[qbuf], ssem)
            if b == 0:
                r1 = base + lax.min(g + 1, ngrp - 1) * GRP
                pltpu.sync_copy(src2.at[pl.ds(r1, GRP)], src_i.at[qg])
                pltpu.sync_copy(dst2.at[pl.ds(r1, GRP)], dst_i.at[qg])
            pltpu.async_copy(bufs.at[pbuf], acc.at[dst_i.at[pg, b]], ssem,
                             add=True)
            if b < GRP - 1:
                pltpu.async_copy(table.at[src_i.at[pg, b + 1]],
                                 bufs.at[qbuf], gsem)
            else:
                pltpu.async_copy(table.at[src_i.at[qg, 0]],
                                 bufs.at[qbuf], gsem)
        return carry
    lax.fori_loop(0, ngrp, body, 0)
    _drain(zeros_hbm, bufs.at[0], ssem)
    _drain(zeros_hbm, bufs.at[1], gsem)


def _make_deg():
    W = 16
    mesh = plsc.VectorSubcoreMesh(core_axis_name="c", subcore_axis_name="s")

    @functools.partial(
        pl.kernel, mesh=mesh,
        compiler_params=pltpu.CompilerParams(use_tc_tiling_on_sc=False),
        out_type=[jax.ShapeDtypeStruct((NP, W), _f32),
                  jax.ShapeDtypeStruct((NP, W), _f32)],
        scratch_types=[pltpu.VMEM((2, GRP, CH), jnp.int32),
                       pltpu.VMEM((CH, W), _f32),
                       pltpu.VMEM((CH, W), _f32),
                       pltpu.VMEM_SHARED((NP, W), _f32),
                       pltpu.SemaphoreType.DMA],
    )
    def deg_k(dst2, ones_hbm, zeros_hbm, out_a, out_b, dst_i, ones_b, dump,
              acc, ssem):
        c = lax.axis_index("c")
        s = lax.axis_index("s")
        pltpu.sync_copy(ones_hbm, ones_b)
        _zero_acc(zeros_hbm, acc, s)
        plsc.subcore_barrier()
        base = (c * 16 + s) * (NCHUNK // 32)
        ngrp = (NCHUNK // 32) // GRP

        pltpu.sync_copy(dst2.at[pl.ds(base, GRP)], dst_i.at[0])
        pltpu.async_copy(zeros_hbm.at[pl.ds(0, CH)], dump, ssem)

        def grp_body(g, carry):
            pg = lax.rem(g, 2)
            qg = lax.rem(g + 1, 2)
            for b in range(GRP):
                _drain(zeros_hbm, dump, ssem)
                if b == 0:
                    r1 = base + lax.min(g + 1, ngrp - 1) * GRP
                    pltpu.sync_copy(dst2.at[pl.ds(r1, GRP)], dst_i.at[qg])
                pltpu.async_copy(ones_b, acc.at[dst_i.at[pg, b]], ssem,
                                 add=True)
            return carry
        lax.fori_loop(0, ngrp, grp_body, 0)
        _drain(zeros_hbm, dump, ssem)
        plsc.subcore_barrier()

        @pl.when(c == 0)
        def _():
            _writeback(acc, out_a, s)

        @pl.when(c == 1)
        def _():
            _writeback(acc, out_b, s)

    return deg_k


def _make_prop16():
    W = 16
    mesh = plsc.VectorSubcoreMesh(core_axis_name="c", subcore_axis_name="s")

    @functools.partial(
        pl.kernel, mesh=mesh,
        compiler_params=pltpu.CompilerParams(use_tc_tiling_on_sc=False),
        out_type=[jax.ShapeDtypeStruct((NP, W), _f32),
                  jax.ShapeDtypeStruct((NP, W), _f32)],
        scratch_types=[pltpu.VMEM((2, GRP, CH), jnp.int32),
                       pltpu.VMEM((2, GRP, CH), jnp.int32),
                       pltpu.VMEM((2, CH, W), _f32),
                       pltpu.VMEM_SHARED((NP, W), _f32),
                       pltpu.SemaphoreType.DMA,
                       pltpu.SemaphoreType.DMA],
    )
    def prop_k(src2, dst2, table, zeros_hbm, out_a, out_b,
               src_i, dst_i, bufs, acc, gsem, ssem):
        c = lax.axis_index("c")
        s = lax.axis_index("s")
        _zero_acc(zeros_hbm, acc, s)
        plsc.subcore_barrier()
        base = (c * 16 + s) * (NCHUNK // 32)
        _acc_ring(src2, dst2, table, zeros_hbm, src_i, dst_i, bufs, acc,
                  gsem, ssem, base, (NCHUNK // 32) // GRP)
        plsc.subcore_barrier()

        @pl.when(c == 0)
        def _():
            _writeback(acc, out_a, s)

        @pl.when(c == 1)
        def _():
            _writeback(acc, out_b, s)

    return prop_k


def _make_prop32():
    W = 32
    mesh = plsc.VectorSubcoreMesh(core_axis_name="c", subcore_axis_name="s")

    @functools.partial(
        pl.kernel, mesh=mesh,
        compiler_params=pltpu.CompilerParams(use_tc_tiling_on_sc=False),
        out_type=[jax.ShapeDtypeStruct((NP, W), _f32),
                  jax.ShapeDtypeStruct((NP, W), _f32)],
        scratch_types=[pltpu.VMEM((2, GRP, CH), jnp.int32),
                       pltpu.VMEM((2, GRP, CH), jnp.int32),
                       pltpu.VMEM((2, CH, W), _f32),
                       pltpu.VMEM_SHARED((NP, W), _f32),
                       pltpu.SemaphoreType.DMA,
                       pltpu.SemaphoreType.DMA],
    )
    def prop_k(src2, dst2, t_lo, t_hi, zeros_hbm, out_lo, out_hi,
               src_i, dst_i, bufs, acc, gsem, ssem):
        c = lax.axis_index("c")
        s = lax.axis_index("s")
        _zero_acc(zeros_hbm, acc, s)
        plsc.subcore_barrier()
        base = s * (NCHUNK // 16)
        ngrp = (NCHUNK // 16) // GRP

        @pl.when(c == 0)
        def _():
            _acc_ring(src2, dst2, t_lo, zeros_hbm, src_i, dst_i, bufs, acc,
                      gsem, ssem, base, ngrp)

        @pl.when(c == 1)
        def _():
            _acc_ring(src2, dst2, t_hi, zeros_hbm, src_i, dst_i, bufs, acc,
                      gsem, ssem, base, ngrp)
        plsc.subcore_barrier()

        @pl.when(c == 0)
        def _():
            _writeback(acc, out_lo, s)

        @pl.when(c == 1)
        def _():
            _writeback(acc, out_hi, s)

    return prop_k


_deg = _make_deg()
_prop16 = _make_prop16()
_prop32 = _make_prop32()


def _row_spec(w):
    return pl.BlockSpec((BR, w), lambda i: (i, 0))


def _full_spec(shape):
    return pl.BlockSpec(shape, lambda i: (0, 0))


def _prep(deg_a, deg_b, xp):
    def body(da, db, xr, g0, dinv):
        deg = da[:, 0:1] + db[:, 0:1] + 1.0
        di = lax.rsqrt(deg)
        dinv[...] = di
        g0[...] = xr[...] * di

    return pl.pallas_call(
        body, grid=(GRID,),
        in_specs=[_row_spec(16), _row_spec(16), _row_spec(16)],
        out_specs=[_row_spec(16), _row_spec(1)],
        out_shape=[jax.ShapeDtypeStruct((N, 16), _f32),
                   jax.ShapeDtypeStruct((N, 1), _f32)],
    )(deg_a, deg_b, xp)


def _layer1(a0a, a0b, g0, dinv, w1p, b1, w2):
    def body(aa, ab, g0r, dr, w1r, b1r, w2r, lo, hi):
        di = dr[...]
        q = di * (aa[...] + ab[...] + g0r[...])
        h1 = jnp.maximum(
            jnp.dot(q, w1r[...], preferred_element_type=_f32) + b1r[...], 0.0)
        g1 = di * jnp.dot(h1, w2r[...], preferred_element_type=_f32)
        lo[...] = g1[:, :32]
        hi[...] = g1[:, 32:]

    return pl.pallas_call(
        body, grid=(GRID,),
        in_specs=[_row_spec(16), _row_spec(16), _row_spec(16), _row_spec(1),
                  _full_spec((16, 64)), _full_spec((1, 64)),
                  _full_spec((64, 64))],
        out_specs=[_row_spec(32), _row_spec(32)],
        out_shape=[jax.ShapeDtypeStruct((N, 32), _f32),
                   jax.ShapeDtypeStruct((N, 32), _f32)],
    )(a0a, a0b, g0, dinv, w1p, b1, w2)


def _layer_mid(alo, ahi, glo, ghi, dinv, b, w):
    def body(alr, ahr, glr, ghr, dr, br, wr, lo, hi):
        di = dr[...]
        h = jnp.concatenate([alr[...] + glr[...], ahr[...] + ghr[...]],
                            axis=1)
        h = jnp.maximum(di * h + br[...], 0.0)
        g = di * jnp.dot(h, wr[...], preferred_element_type=_f32)
        lo[...] = g[:, :32]
        hi[...] = g[:, 32:]

    return pl.pallas_call(
        body, grid=(GRID,),
        in_specs=[_row_spec(32), _row_spec(32), _row_spec(32), _row_spec(32),
                  _row_spec(1), _full_spec((1, 64)), _full_spec((64, 64))],
        out_specs=[_row_spec(32), _row_spec(32)],
        out_shape=[jax.ShapeDtypeStruct((N, 32), _f32),
                   jax.ShapeDtypeStruct((N, 32), _f32)],
    )(alo, ahi, glo, ghi, dinv, b, w)


def _final(alo, ahi, glo, ghi, dinv, b3, xp, wf1p, bf1, wf2, bf2, wcat, bcat):
    def body(alr, ahr, glr, ghr, dr, b3r, xr, wf1r, bf1r, wf2r, bf2r,
             wcr, bcr, out, emb_s, gfs):
        i = pl.program_id(0)

        @pl.when(i == 0)
        def _():
            emb_s[...] = jnp.zeros_like(emb_s)
            for k in range(6):
                gfs[k] = 0.0

        di = dr[...]
        h = jnp.concatenate([alr[...] + glr[...], ahr[...] + ghr[...]],
                            axis=1)
        h3 = jnp.maximum(di * h + b3r[...], 0.0)
        emb_s[...] = emb_s[...] + jnp.sum(h3, axis=0, keepdims=True)

        xb = xr[...]
        x2 = xb[:, 2:3]
        m = x2 == 1.0
        gfs[0] = gfs[0] + jnp.sum(x2)
        gfs[1] = gfs[1] + jnp.sum(xb[:, 3:4])
        gfs[2] = gfs[2] + jnp.sum(xb[:, 4:5])
        gfs[3] = gfs[3] + jnp.sum(m.astype(_f32))
        gfs[4] = gfs[4] + jnp.sum(jnp.where(m, xb[:, 0:1], 0.0))
        gfs[5] = gfs[5] + jnp.sum(jnp.where(m, xb[:, 1:2], 0.0))

        @pl.when(i == GRID - 1)
        def _():
            emb = emb_s[...] * (1.0 / N)
            n_comp, n_and, n_or = gfs[0], gfs[1], gfs[2]
            cnt, s0, s1 = gfs[3], gfs[4], gfs[5]
            avg_l = jnp.where(cnt > 0, s0 / jnp.maximum(cnt, 1.0), 0.0)
            avg_m = jnp.where(cnt > 0, s1 / jnp.maximum(cnt, 1.0), 0.0)
            lane = lax.broadcasted_iota(jnp.int32, (1, 8), 1)
            z = jnp.zeros((1, 8), _f32)
            gf = (jnp.where(lane == 0, n_comp, z)
                  + jnp.where(lane == 1, n_and, z)
                  + jnp.where(lane == 2, n_or, z)
                  + jnp.where(lane == 3, n_and + n_or, z)
                  + jnp.where(lane == 4, avg_l, z)
                  + jnp.where(lane == 5, avg_m, z)
                  + jnp.where(lane == 6, jnp.float32(0.2), z))
            f1 = jnp.maximum(
                jnp.dot(emb, wf1r[0:64, :], preferred_element_type=_f32)
                + jnp.dot(gf, wf1r[64:72, :], preferred_element_type=_f32)
                + bf1r[...], 0.0)
            f2 = jnp.maximum(
                jnp.dot(f1, wf2r[...], preferred_element_type=_f32)
                + bf2r[...], 0.0)
            out[...] = (jnp.dot(f2, wcr[...], preferred_element_type=_f32)
                        + bcr[...])

    return pl.pallas_call(
        body, grid=(GRID,),
        in_specs=[_row_spec(32), _row_spec(32), _row_spec(32), _row_spec(32),
                  _row_spec(1), _full_spec((1, 64)), _row_spec(16),
                  _full_spec((72, 64)), _full_spec((1, 64)),
                  _full_spec((64, 32)), _full_spec((1, 32)),
                  _full_spec((32, 16)), _full_spec((1, 16))],
        out_specs=pl.BlockSpec((1, 16), lambda i: (0, 0)),
        out_shape=jax.ShapeDtypeStruct((1, 16), _f32),
        scratch_shapes=[pltpu.VMEM((1, 64), _f32),
                        pltpu.SMEM((8,), _f32)],
    )(alo, ahi, glo, ghi, dinv, b3, xp, wf1p, bf1, wf2, bf2, wcat, bcat)


def kernel(x, edge_index, W1, b1, W2, b2, W3, b3,
           Wf1, bf1, Wf2, bf2, Wis, bis, Wmc, bmc):
    xp = jnp.pad(x, ((0, 0), (0, 11)))
    src = jnp.concatenate(
        [edge_index[0], jnp.zeros((EP - E,), jnp.int32)])
    pad_dst = N + (jnp.arange(EP - E, dtype=jnp.int32) % (NP - N))
    dst = jnp.concatenate([edge_index[1], pad_dst])
    src2 = src.reshape(NCHUNK, CH)
    dst2 = dst.reshape(NCHUNK, CH)
    ones16 = jnp.ones((CH, 16), _f32)
    zeros16 = jnp.zeros((ZR, 16), _f32)
    zeros32 = jnp.zeros((ZR, 32), _f32)

    deg_a, deg_b = _deg(dst2, ones16, zeros16)
    g0, dinv = _prep(deg_a, deg_b, xp)
    a0a, a0b = _prop16(src2, dst2, g0, zeros16)
    w1p = jnp.pad(W1, ((0, 11), (0, 0)))
    g1lo, g1hi = _layer1(a0a, a0b, g0, dinv, w1p, b1.reshape(1, -1), W2)
    a1lo, a1hi = _prop32(src2, dst2, g1lo, g1hi, zeros32)
    g2lo, g2hi = _layer_mid(a1lo, a1hi, g1lo, g1hi, dinv,
                            b2.reshape(1, -1), W3)
    a2lo, a2hi = _prop32(src2, dst2, g2lo, g2hi, zeros32)
    wf1p = jnp.pad(Wf1, ((0, 1), (0, 0)))
    wcat = jnp.concatenate([jnp.pad(Wis, ((0, 0), (0, 2))),
                            jnp.pad(Wmc, ((0, 0), (0, 2)))], axis=1)
    bcat = jnp.concatenate([jnp.pad(bis, (0, 2)),
                            jnp.pad(bmc, (0, 2))]).reshape(1, 16)
    out = _final(a2lo, a2hi, g2lo, g2hi, dinv, b3.reshape(1, -1), xp,
                 wf1p, bf1.reshape(1, -1), Wf2, bf2.reshape(1, -1),
                 wcat, bcat)
    return out[:, 0:6], out[:, 8:14]
